# baseline (speedup 1.0000x reference)
import functools

import jax
import jax.numpy as jnp
from jax import lax
from jax.experimental import pallas as pl
from jax.experimental.pallas import tpu as pltpu
from jax.experimental.pallas import tpu_sc as plsc

_NC = 2
_NS = 16
_NW = _NC * _NS

_SEED = 16
_BLOCK = _SEED * _NS
_SP = 10


def _make_broadcast_kernel(S, D, dtype):
    b_per_w = S // _NW
    n_dma = b_per_w // _BLOCK
    mesh = plsc.VectorSubcoreMesh(core_axis_name="c", subcore_axis_name="s")

    @functools.partial(
        pl.kernel,
        out_type=jax.ShapeDtypeStruct((S, D), dtype),
        mesh=mesh,
        scratch_types=[
            pltpu.VMEM((_SEED,), jnp.int32),
            pltpu.VMEM((_SEED, D), dtype),
            pltpu.VMEM_SHARED((_BLOCK, D), dtype),
            pltpu.SemaphoreType.DMA,
            pltpu.SemaphoreType.DMA,
        ],
    )
    def broadcast_kernel(table_hbm, idx_hbm, out_hbm, idx_v, row_v, shared_v,
                         gsem, wsem):
        cid = lax.axis_index("c")
        sid = lax.axis_index("s")
        base = (cid * _NS + sid) * b_per_w

        @pl.when(sid == 0)
        def _():
            pltpu.sync_copy(idx_hbm, idx_v)
            pltpu.async_copy(table_hbm.at[idx_v], row_v, gsem).wait()
            pltpu.sync_copy(row_v, shared_v.at[pl.ds(0, _SEED)])

        plsc.subcore_barrier()

        seed = shared_v.at[pl.ds(0, _SEED)]
        n_chunks = b_per_w // _SEED
        spmem_share = [(j * _SP) // n_chunks != ((j + 1) * _SP) // n_chunks
                       for j in range(n_chunks)]
        copies = [
            pltpu.async_copy(
                seed, out_hbm.at[pl.ds(base + j * _SEED, _SEED)], wsem
            )
            for j, use_spmem in enumerate(spmem_share) if use_spmem
        ]
        @pl.when(sid > 0)
        def _():
            pltpu.sync_copy(seed, row_v)
        copies += [
            pltpu.async_copy(
                row_v, out_hbm.at[pl.ds(base + j * _SEED, _SEED)], gsem
            )
            for j, use_spmem in enumerate(spmem_share) if not use_spmem
        ]
        for c in copies:
            c.wait()

    return broadcast_kernel


def kernel(embeddings, modality_embedding, token_type_id):
    S = embeddings.shape[1]
    D = modality_embedding.shape[1]
    idx = jnp.full((_SEED,), token_type_id, dtype=jnp.int32)
    fn = _make_broadcast_kernel(S, D, modality_embedding.dtype)
    return fn(modality_embedding, idx)


# speedup vs baseline: 1.0522x; 1.0522x over previous
import functools

import jax
import jax.numpy as jnp
from jax import lax
from jax.experimental import pallas as pl
from jax.experimental.pallas import tpu as pltpu
from jax.experimental.pallas import tpu_sc as plsc

_NC = 2
_NS = 16
_NW = _NC * _NS

_SEED = 16
_BLOCK = _SEED * _NS
_SP = 8


def _make_broadcast_kernel(S, D, dtype):
    b_per_w = S // _NW
    n_dma = b_per_w // _BLOCK
    mesh = plsc.VectorSubcoreMesh(core_axis_name="c", subcore_axis_name="s")

    @functools.partial(
        pl.kernel,
        out_type=jax.ShapeDtypeStruct((S, D), dtype),
        mesh=mesh,
        scratch_types=[
            pltpu.VMEM((_SEED,), jnp.int32),
            pltpu.VMEM((_SEED, D), dtype),
            pltpu.VMEM_SHARED((_BLOCK, D), dtype),
            pltpu.SemaphoreType.DMA,
            pltpu.SemaphoreType.DMA,
        ],
    )
    def broadcast_kernel(table_hbm, idx_hbm, out_hbm, idx_v, row_v, shared_v,
                         gsem, wsem):
        cid = lax.axis_index("c")
        sid = lax.axis_index("s")
        base = (cid * _NS + sid) * b_per_w

        @pl.when(sid == 0)
        def _():
            pltpu.sync_copy(idx_hbm, idx_v)
            pltpu.async_copy(table_hbm.at[idx_v], row_v, gsem).wait()
            pltpu.sync_copy(row_v, shared_v.at[pl.ds(0, _SEED)])

        plsc.subcore_barrier()

        seed = shared_v.at[pl.ds(0, _SEED)]
        n_chunks = b_per_w // _SEED
        spmem_share = [(j * _SP) // n_chunks != ((j + 1) * _SP) // n_chunks
                       for j in range(n_chunks)]
        copies = [
            pltpu.async_copy(
                seed, out_hbm.at[pl.ds(base + j * _SEED, _SEED)], wsem
            )
            for j, use_spmem in enumerate(spmem_share) if use_spmem
        ]
        @pl.when(sid > 0)
        def _():
            pltpu.sync_copy(seed, row_v)
        copies += [
            pltpu.async_copy(
                row_v, out_hbm.at[pl.ds(base + j * _SEED, _SEED)], gsem
            )
            for j, use_spmem in enumerate(spmem_share) if not use_spmem
        ]
        for c in copies:
            c.wait()

    return broadcast_kernel


def kernel(embeddings, modality_embedding, token_type_id):
    S = embeddings.shape[1]
    D = modality_embedding.shape[1]
    idx = jnp.full((_SEED,), token_type_id, dtype=jnp.int32)
    fn = _make_broadcast_kernel(S, D, modality_embedding.dtype)
    return fn(modality_embedding, idx)


# speedup vs baseline: 1.0738x; 1.0206x over previous
import functools

import jax
import jax.numpy as jnp
from jax import lax
from jax.experimental import pallas as pl
from jax.experimental.pallas import tpu as pltpu
from jax.experimental.pallas import tpu_sc as plsc

_NC = 2
_NS = 16
_NW = _NC * _NS

_SEED = 16
_BLOCK = _SEED * _NS
_SP = 7


def _make_broadcast_kernel(S, D, dtype):
    b_per_w = S // _NW
    n_dma = b_per_w // _BLOCK
    mesh = plsc.VectorSubcoreMesh(core_axis_name="c", subcore_axis_name="s")

    @functools.partial(
        pl.kernel,
        out_type=jax.ShapeDtypeStruct((S, D), dtype),
        mesh=mesh,
        scratch_types=[
            pltpu.VMEM((_SEED,), jnp.int32),
            pltpu.VMEM((_SEED, D), dtype),
            pltpu.VMEM_SHARED((_BLOCK, D), dtype),
            pltpu.SemaphoreType.DMA,
            pltpu.SemaphoreType.DMA,
        ],
    )
    def broadcast_kernel(table_hbm, idx_hbm, out_hbm, idx_v, row_v, shared_v,
                         gsem, wsem):
        cid = lax.axis_index("c")
        sid = lax.axis_index("s")
        base = (cid * _NS + sid) * b_per_w

        @pl.when(sid == 0)
        def _():
            pltpu.sync_copy(idx_hbm, idx_v)
            pltpu.async_copy(table_hbm.at[idx_v], row_v, gsem).wait()
            pltpu.sync_copy(row_v, shared_v.at[pl.ds(0, _SEED)])

        plsc.subcore_barrier()

        seed = shared_v.at[pl.ds(0, _SEED)]
        n_chunks = b_per_w // _SEED
        spmem_share = [(j * _SP) // n_chunks != ((j + 1) * _SP) // n_chunks
                       for j in range(n_chunks)]
        copies = [
            pltpu.async_copy(
                seed, out_hbm.at[pl.ds(base + j * _SEED, _SEED)], wsem
            )
            for j, use_spmem in enumerate(spmem_share) if use_spmem
        ]
        @pl.when(sid > 0)
        def _():
            pltpu.sync_copy(seed, row_v)
        copies += [
            pltpu.async_copy(
                row_v, out_hbm.at[pl.ds(base + j * _SEED, _SEED)], gsem
            )
            for j, use_spmem in enumerate(spmem_share) if not use_spmem
        ]
        for c in copies:
            c.wait()

    return broadcast_kernel


def kernel(embeddings, modality_embedding, token_type_id):
    S = embeddings.shape[1]
    D = modality_embedding.shape[1]
    idx = jnp.full((_SEED,), token_type_id, dtype=jnp.int32)
    fn = _make_broadcast_kernel(S, D, modality_embedding.dtype)
    return fn(modality_embedding, idx)


# speedup vs baseline: 1.0762x; 1.0022x over previous
import functools

import jax
import jax.numpy as jnp
from jax import lax
from jax.experimental import pallas as pl
from jax.experimental.pallas import tpu as pltpu
from jax.experimental.pallas import tpu_sc as plsc

_NC = 2
_NS = 16
_NW = _NC * _NS

_SEED = 16
_BLOCK = _SEED * _NS
_SP = 6


def _make_broadcast_kernel(S, D, dtype):
    b_per_w = S // _NW
    n_dma = b_per_w // _BLOCK
    mesh = plsc.VectorSubcoreMesh(core_axis_name="c", subcore_axis_name="s")

    @functools.partial(
        pl.kernel,
        out_type=jax.ShapeDtypeStruct((S, D), dtype),
        mesh=mesh,
        scratch_types=[
            pltpu.VMEM((_SEED,), jnp.int32),
            pltpu.VMEM((_SEED, D), dtype),
            pltpu.VMEM_SHARED((_BLOCK, D), dtype),
            pltpu.SemaphoreType.DMA,
            pltpu.SemaphoreType.DMA,
        ],
    )
    def broadcast_kernel(table_hbm, idx_hbm, out_hbm, idx_v, row_v, shared_v,
                         gsem, wsem):
        cid = lax.axis_index("c")
        sid = lax.axis_index("s")
        base = (cid * _NS + sid) * b_per_w

        @pl.when(sid == 0)
        def _():
            pltpu.sync_copy(idx_hbm, idx_v)
            pltpu.async_copy(table_hbm.at[idx_v], row_v, gsem).wait()
            pltpu.sync_copy(row_v, shared_v.at[pl.ds(0, _SEED)])

        plsc.subcore_barrier()

        seed = shared_v.at[pl.ds(0, _SEED)]
        n_chunks = b_per_w // _SEED
        spmem_share = [(j * _SP) // n_chunks != ((j + 1) * _SP) // n_chunks
                       for j in range(n_chunks)]
        copies = [
            pltpu.async_copy(
                seed, out_hbm.at[pl.ds(base + j * _SEED, _SEED)], wsem
            )
            for j, use_spmem in enumerate(spmem_share) if use_spmem
        ]
        @pl.when(sid > 0)
        def _():
            pltpu.sync_copy(seed, row_v)
        copies += [
            pltpu.async_copy(
                row_v, out_hbm.at[pl.ds(base + j * _SEED, _SEED)], gsem
            )
            for j, use_spmem in enumerate(spmem_share) if not use_spmem
        ]
        for c in copies:
            c.wait()

    return broadcast_kernel


def kernel(embeddings, modality_embedding, token_type_id):
    S = embeddings.shape[1]
    D = modality_embedding.shape[1]
    idx = jnp.full((_SEED,), token_type_id, dtype=jnp.int32)
    fn = _make_broadcast_kernel(S, D, modality_embedding.dtype)
    return fn(modality_embedding, idx)


# speedup vs baseline: 1.0768x; 1.0005x over previous
import functools

import jax
import jax.numpy as jnp
from jax import lax
from jax.experimental import pallas as pl
from jax.experimental.pallas import tpu as pltpu
from jax.experimental.pallas import tpu_sc as plsc

_NC = 2
_NS = 16
_NW = _NC * _NS

_SEED = 16
_BLOCK = _SEED * _NS
_SP = 5


def _make_broadcast_kernel(S, D, dtype):
    b_per_w = S // _NW
    n_dma = b_per_w // _BLOCK
    mesh = plsc.VectorSubcoreMesh(core_axis_name="c", subcore_axis_name="s")

    @functools.partial(
        pl.kernel,
        out_type=jax.ShapeDtypeStruct((S, D), dtype),
        mesh=mesh,
        scratch_types=[
            pltpu.VMEM((_SEED,), jnp.int32),
            pltpu.VMEM((_SEED, D), dtype),
            pltpu.VMEM_SHARED((_BLOCK, D), dtype),
            pltpu.SemaphoreType.DMA,
            pltpu.SemaphoreType.DMA,
        ],
    )
    def broadcast_kernel(table_hbm, idx_hbm, out_hbm, idx_v, row_v, shared_v,
                         gsem, wsem):
        cid = lax.axis_index("c")
        sid = lax.axis_index("s")
        base = (cid * _NS + sid) * b_per_w

        @pl.when(sid == 0)
        def _():
            pltpu.sync_copy(idx_hbm, idx_v)
            pltpu.async_copy(table_hbm.at[idx_v], row_v, gsem).wait()
            pltpu.sync_copy(row_v, shared_v.at[pl.ds(0, _SEED)])

        plsc.subcore_barrier()

        seed = shared_v.at[pl.ds(0, _SEED)]
        n_chunks = b_per_w // _SEED
        spmem_share = [(j * _SP) // n_chunks != ((j + 1) * _SP) // n_chunks
                       for j in range(n_chunks)]
        copies = [
            pltpu.async_copy(
                seed, out_hbm.at[pl.ds(base + j * _SEED, _SEED)], wsem
            )
            for j, use_spmem in enumerate(spmem_share) if use_spmem
        ]
        @pl.when(sid > 0)
        def _():
            pltpu.sync_copy(seed, row_v)
        copies += [
            pltpu.async_copy(
                row_v, out_hbm.at[pl.ds(base + j * _SEED, _SEED)], gsem
            )
            for j, use_spmem in enumerate(spmem_share) if not use_spmem
        ]
        for c in copies:
            c.wait()

    return broadcast_kernel


def kernel(embeddings, modality_embedding, token_type_id):
    S = embeddings.shape[1]
    D = modality_embedding.shape[1]
    idx = jnp.full((_SEED,), token_type_id, dtype=jnp.int32)
    fn = _make_broadcast_kernel(S, D, modality_embedding.dtype)
    return fn(modality_embedding, idx)
